# baseline (speedup 1.0000x reference)
import functools

import jax
import jax.numpy as jnp
from jax import lax
from jax.experimental import pallas as pl
from jax.experimental.pallas import tpu as pltpu
from jax.experimental.pallas import tpu_sc as plsc

_NC = 2
_NS = 16
_NW = _NC * _NS
_L = 16

_CH = 64
_RCH = 8192


def _make_fused(max_len, d_half, n):
    rows_w = n // _NW
    n_pairs = rows_w // (2 * _CH)
    assert rows_w % (2 * _CH) == 0
    d_out = 2 * d_half
    xg_len = 2 * rows_w
    xr_len = 2 * n // _NS
    n_rch = xr_len // _RCH
    assert xr_len % _RCH == 0 and n_rch % 2 == 0
    scale_max = float(max_len - 1)
    mesh = plsc.VectorSubcoreMesh(core_axis_name="c", subcore_axis_name="s")

    @functools.partial(
        pl.kernel,
        mesh=mesh,
        out_type=jax.ShapeDtypeStruct((n * d_out,), jnp.float32),
        compiler_params=pltpu.CompilerParams(needs_layout_passes=False),
        scratch_types=[
            pltpu.VMEM((max_len * d_half,), jnp.float32),
            pltpu.VMEM((xg_len,), jnp.float32),
            pltpu.VMEM((2 * _RCH,), jnp.float32),
            pltpu.VMEM((_CH * d_out,), jnp.float32),
            pltpu.VMEM((_CH * d_out,), jnp.float32),
            pltpu.VMEM((2 * _CH,), jnp.int32),
            pltpu.VMEM((2 * _L,), jnp.float32),
            pltpu.VMEM((_NS * 2 * _L,), jnp.float32),
            pltpu.VMEM_SHARED((_NS * 2 * _L,), jnp.float32),
            pltpu.SemaphoreType.DMA,
            pltpu.SemaphoreType.DMA,
            pltpu.SemaphoreType.DMA,
            pltpu.SemaphoreType.DMA,
            pltpu.SemaphoreType.DMA,
            pltpu.SemaphoreType.DMA,
        ],
    )
    def fused(
        x_hbm,
        pe_hbm,
        out_hbm,
        pe_v,
        xg_v,
        xr_v,
        out_v0,
        out_v1,
        idx_s,
        red_v,
        all_v,
        shared,
        sem_pe,
        sem_xg,
        sem_r0,
        sem_r1,
        sem_o0,
        sem_o1,
    ):
        sid = lax.axis_index("s")
        wid = sid * _NC + lax.axis_index("c")
        row0 = wid * rows_w
        pe_cp = pltpu.async_copy(pe_hbm, pe_v, sem_pe)
        xg_cp = pltpu.async_copy(
            x_hbm.at[pl.ds(2 * row0, xg_len)], xg_v, sem_xg
        )

        iota = lax.iota(jnp.int32, _L)
        even = (iota % 2) == 0
        inf = jnp.float32(jnp.inf)

        rbase = sid * xr_len
        rsems = (sem_r0, sem_r1)

        def rchunk(rc):
            return x_hbm.at[pl.ds(rbase + rc * _RCH, _RCH)]

        pltpu.async_copy(rchunk(0), xr_v.at[pl.ds(0, _RCH)], sem_r0)
        pltpu.async_copy(rchunk(1), xr_v.at[pl.ds(_RCH, _RCH)], sem_r1)
        mn = jnp.full((_L,), inf)
        mx = jnp.full((_L,), -inf)
        for rc in range(n_rch):
            b = rc % 2
            pltpu.make_async_copy(
                rchunk(rc), xr_v.at[pl.ds(b * _RCH, _RCH)], rsems[b]
            ).wait()

            def red(i, mnmx):
                a, z = mnmx
                v = xr_v[pl.ds(b * _RCH + i * _L, _L)]
                return jnp.minimum(a, v), jnp.maximum(z, v)

            mn, mx = lax.fori_loop(0, _RCH // _L, red, (mn, mx))
            if rc + 2 < n_rch:
                pltpu.async_copy(
                    rchunk(rc + 2), xr_v.at[pl.ds(b * _RCH, _RCH)], rsems[b]
                )

        red_v[pl.ds(0, _L)] = mn
        red_v[pl.ds(_L, _L)] = mx
        pltpu.sync_copy(red_v, shared.at[pl.ds(sid * 2 * _L, 2 * _L)])
        plsc.subcore_barrier()
        pltpu.sync_copy(shared, all_v)

        def red2(w, mnmx):
            a, z = mnmx
            p = all_v[pl.ds(w * 2 * _L, _L)]
            q = all_v[pl.ds(w * 2 * _L + _L, _L)]
            return jnp.minimum(a, p), jnp.maximum(z, q)

        mn, mx = lax.fori_loop(
            0, _NS, red2, (jnp.full((_L,), inf), jnp.full((_L,), -inf))
        )
        mnx = jnp.min(jnp.where(even, mn, inf))
        mny = jnp.min(jnp.where(even, inf, mn))
        mxx = jnp.max(jnp.where(even, mx, -inf))
        mxy = jnp.max(jnp.where(even, -inf, mx))
        mnv = jnp.where(even, mnx, mny)
        dnv = jnp.where(even, mxx - mnx, mxy - mny) + 1e-8

        def do_chunk(c, out_v, sem):
            def mkidx(g, carry):
                v = xg_v[pl.ds((2 * c * _CH) + g * _L, _L)]
                xn = jnp.clip((v - mnv) / dnv, 0.0, 1.0)
                idx_s[pl.ds(g * _L, _L)] = (xn * scale_max).astype(
                    jnp.int32
                ) * d_half
                return carry

            lax.fori_loop(0, 2 * _CH // _L, mkidx, 0)

            @pl.when(c >= 2)
            def _():
                pltpu.make_async_copy(
                    out_v,
                    out_hbm.at[pl.ds((row0 + c * _CH) * d_out, _CH * d_out)],
                    sem,
                ).wait()

            @plsc.parallel_loop(0, 2 * _CH // _L, unroll=2)
            def group(g):
                iv = idx_s[pl.ds(g * _L, _L)]
                for k in range(8):
                    ix = iv[2 * k]
                    iy = iv[2 * k + 1]
                    ob = (g * 8 + k) * d_out
                    for cc in range(0, d_half, _L):
                        out_v[pl.ds(ob + cc, _L)] = pe_v[pl.ds(ix + cc, _L)]
                    for cc in range(0, d_half, _L):
                        out_v[pl.ds(ob + d_half + cc, _L)] = pe_v[
                            pl.ds(iy + cc, _L)
                        ]

            pltpu.async_copy(
                out_v,
                out_hbm.at[pl.ds((row0 + c * _CH) * d_out, _CH * d_out)],
                sem,
            )

        pe_cp.wait()
        xg_cp.wait()

        def pair(ci, carry):
            do_chunk(2 * ci, out_v0, sem_o0)
            do_chunk(2 * ci + 1, out_v1, sem_o1)
            return carry

        lax.fori_loop(0, n_pairs, pair, 0)

        last = 2 * n_pairs - 1
        pltpu.make_async_copy(
            out_v0,
            out_hbm.at[pl.ds((row0 + (last - 1) * _CH) * d_out, _CH * d_out)],
            sem_o0,
        ).wait()
        pltpu.make_async_copy(
            out_v1,
            out_hbm.at[pl.ds((row0 + last * _CH) * d_out, _CH * d_out)],
            sem_o1,
        ).wait()

    return fused


def kernel(x, pe):
    n, two = x.shape
    max_len, d_half = pe.shape
    out_flat = _make_fused(max_len, d_half, n)(x.reshape(-1), pe.reshape(-1))
    return out_flat.reshape(n, 2 * d_half)


# speedup vs baseline: 1.1834x; 1.1834x over previous
import functools

import jax
import jax.numpy as jnp
from jax import lax
from jax.experimental import pallas as pl
from jax.experimental.pallas import tpu as pltpu
from jax.experimental.pallas import tpu_sc as plsc

_NC = 2
_NS = 16
_NW = _NC * _NS
_L = 16

_CH = 128


def _index_body(x_ref, idx_ref, *, scale_max, stride):
    v = x_ref[...]
    lane = lax.broadcasted_iota(jnp.int32, (1, v.shape[1]), 1)
    even = (lane % 2) == 0
    colmin = jnp.min(v, axis=0, keepdims=True)
    colmax = jnp.max(v, axis=0, keepdims=True)
    mn0 = jnp.min(jnp.where(even, colmin, jnp.inf))
    mn1 = jnp.min(jnp.where(even, jnp.inf, colmin))
    mx0 = jnp.max(jnp.where(even, colmax, -jnp.inf))
    mx1 = jnp.max(jnp.where(even, -jnp.inf, colmax))
    mnv = jnp.where(even, mn0, mn1)
    dnv = jnp.where(even, mx0 - mn0, mx1 - mn1) + 1e-8
    xn = jnp.clip((v - mnv) / dnv, 0.0, 1.0)
    idx_ref[...] = (xn * scale_max).astype(jnp.int32) * stride


def _compute_indices(xr, scale_max, stride):
    return pl.pallas_call(
        functools.partial(_index_body, scale_max=scale_max, stride=stride),
        out_shape=jax.ShapeDtypeStruct(xr.shape, jnp.int32),
    )(xr)


def _make_sc_gather(max_len, d_half, n):
    rows_w = n // _NW
    n_pairs = rows_w // (2 * _CH)
    assert rows_w % (2 * _CH) == 0
    d_out = 2 * d_half
    mesh = plsc.VectorSubcoreMesh(core_axis_name="c", subcore_axis_name="s")

    @functools.partial(
        pl.kernel,
        mesh=mesh,
        out_type=jax.ShapeDtypeStruct((n, d_out), jnp.float32),
        compiler_params=pltpu.CompilerParams(needs_layout_passes=False),
        scratch_types=[
            pltpu.VMEM((max_len * d_half,), jnp.float32),
            pltpu.VMEM((2 * rows_w,), jnp.int32),
            pltpu.VMEM((_CH, d_out), jnp.float32),
            pltpu.VMEM((_CH, d_out), jnp.float32),
            pltpu.SemaphoreType.DMA,
            pltpu.SemaphoreType.DMA,
            pltpu.SemaphoreType.DMA,
            pltpu.SemaphoreType.DMA,
        ],
    )
    def sc_gather(
        pe_hbm,
        idx_hbm,
        out_hbm,
        pe_v,
        idx_v,
        out_v0,
        out_v1,
        sem_pe,
        sem_ix,
        sem_o0,
        sem_o1,
    ):
        wid = lax.axis_index("s") * _NC + lax.axis_index("c")
        row0 = wid * rows_w
        pe_cp = pltpu.async_copy(pe_hbm, pe_v, sem_pe)
        pltpu.async_copy(
            idx_hbm.at[pl.ds(2 * row0, 2 * rows_w)], idx_v, sem_ix
        ).wait()
        pe_cp.wait()

        def do_chunk(c, out_v, sem):
            @pl.when(c >= 2)
            def _():
                pltpu.make_async_copy(
                    out_v, out_hbm.at[pl.ds(row0 + c * _CH, _CH)], sem
                ).wait()

            @plsc.parallel_loop(0, 2 * _CH // _L, unroll=2)
            def group(g):
                iv = idx_v[pl.ds(c * 2 * _CH + g * _L, _L)]
                for k in range(8):
                    ix = iv[2 * k]
                    iy = iv[2 * k + 1]
                    row = g * 8 + k
                    for cc in range(0, d_half, _L):
                        out_v[row, pl.ds(cc, _L)] = pe_v[pl.ds(ix + cc, _L)]
                    for cc in range(0, d_half, _L):
                        out_v[row, pl.ds(d_half + cc, _L)] = pe_v[
                            pl.ds(iy + cc, _L)
                        ]

            pltpu.async_copy(
                out_v, out_hbm.at[pl.ds(row0 + c * _CH, _CH)], sem
            )

        def pair(ci, carry):
            do_chunk(2 * ci, out_v0, sem_o0)
            do_chunk(2 * ci + 1, out_v1, sem_o1)
            return carry

        lax.fori_loop(0, n_pairs, pair, 0)

        last = 2 * n_pairs - 1
        pltpu.make_async_copy(
            out_v0, out_hbm.at[pl.ds(row0 + (last - 1) * _CH, _CH)], sem_o0
        ).wait()
        pltpu.make_async_copy(
            out_v1, out_hbm.at[pl.ds(row0 + last * _CH, _CH)], sem_o1
        ).wait()

    return sc_gather


def kernel(x, pe):
    n, two = x.shape
    max_len, d_half = pe.shape

    xr = x.reshape(n * two // 128, 128)
    idx2d = _compute_indices(xr, float(max_len - 1), d_half)
    idx_flat = idx2d.reshape(n * two)

    return _make_sc_gather(max_len, d_half, n)(pe.reshape(-1), idx_flat)


# speedup vs baseline: 2.2025x; 1.8611x over previous
import functools

import jax
import jax.numpy as jnp
from jax import lax
from jax.experimental import pallas as pl
from jax.experimental.pallas import tpu as pltpu
from jax.experimental.pallas import tpu_sc as plsc

_NC = 2
_NS = 16
_NW = _NC * _NS
_L = 16

_CH = 128


def _index_body(x_ref, idx_ref, *, scale_max, stride):
    def one(v):
        mn = jnp.min(v)
        dn = jnp.max(v) - mn + 1e-8
        xn = jnp.clip((v - mn) / dn, 0.0, 1.0)
        return (xn * scale_max).astype(jnp.int32) * stride

    v = x_ref[...]
    ix = one(v[0])
    iy = one(v[1])
    idx_ref[...] = jnp.concatenate([ix[None], iy[None]], axis=0)


def _compute_indices(xt, scale_max, stride):
    return pl.pallas_call(
        functools.partial(_index_body, scale_max=scale_max, stride=stride),
        out_shape=jax.ShapeDtypeStruct(xt.shape, jnp.int32),
    )(xt)


def _make_sc_gather(max_len, d_half, n):
    rows_w = n // _NW
    n_pairs = rows_w // (2 * _CH)
    assert rows_w % (2 * _CH) == 0
    d_out = 2 * d_half
    mesh = plsc.VectorSubcoreMesh(core_axis_name="c", subcore_axis_name="s")

    @functools.partial(
        pl.kernel,
        mesh=mesh,
        out_type=jax.ShapeDtypeStruct((n, d_out), jnp.float32),
        compiler_params=pltpu.CompilerParams(needs_layout_passes=False),
        scratch_types=[
            pltpu.VMEM((max_len * d_half,), jnp.float32),
            pltpu.VMEM((rows_w,), jnp.int32),
            pltpu.VMEM((rows_w,), jnp.int32),
            pltpu.VMEM((_CH, d_out), jnp.float32),
            pltpu.VMEM((_CH, d_out), jnp.float32),
            pltpu.SemaphoreType.DMA,
            pltpu.SemaphoreType.DMA,
            pltpu.SemaphoreType.DMA,
            pltpu.SemaphoreType.DMA,
        ],
    )
    def sc_gather(
        pe_hbm,
        idx_hbm,
        out_hbm,
        pe_v,
        ixs_v,
        iys_v,
        out_v0,
        out_v1,
        sem_pe,
        sem_ix,
        sem_o0,
        sem_o1,
    ):
        wid = lax.axis_index("s") * _NC + lax.axis_index("c")
        row0 = wid * rows_w
        pe_cp = pltpu.async_copy(pe_hbm, pe_v, sem_pe)
        ix_cp = pltpu.async_copy(idx_hbm.at[pl.ds(row0, rows_w)], ixs_v, sem_ix)
        pltpu.async_copy(
            idx_hbm.at[pl.ds(n + row0, rows_w)], iys_v, sem_ix
        ).wait()
        ix_cp.wait()
        pe_cp.wait()

        def do_chunk(c, out_v, sem):
            @pl.when(c >= 2)
            def _():
                pltpu.make_async_copy(
                    out_v, out_hbm.at[pl.ds(row0 + c * _CH, _CH)], sem
                ).wait()

            @plsc.parallel_loop(0, _CH // _L, unroll=2)
            def group(g):
                ivx = ixs_v[pl.ds(c * _CH + g * _L, _L)]
                ivy = iys_v[pl.ds(c * _CH + g * _L, _L)]
                for k in range(_L):
                    ix = ivx[k]
                    iy = ivy[k]
                    row = g * _L + k
                    for cc in range(0, d_half, _L):
                        out_v[row, pl.ds(cc, _L)] = pe_v[pl.ds(ix + cc, _L)]
                    for cc in range(0, d_half, _L):
                        out_v[row, pl.ds(d_half + cc, _L)] = pe_v[
                            pl.ds(iy + cc, _L)
                        ]

            pltpu.async_copy(
                out_v, out_hbm.at[pl.ds(row0 + c * _CH, _CH)], sem
            )

        def pair(ci, carry):
            do_chunk(2 * ci, out_v0, sem_o0)
            do_chunk(2 * ci + 1, out_v1, sem_o1)
            return carry

        lax.fori_loop(0, n_pairs, pair, 0)

        last = 2 * n_pairs - 1
        pltpu.make_async_copy(
            out_v0, out_hbm.at[pl.ds(row0 + (last - 1) * _CH, _CH)], sem_o0
        ).wait()
        pltpu.make_async_copy(
            out_v1, out_hbm.at[pl.ds(row0 + last * _CH, _CH)], sem_o1
        ).wait()

    return sc_gather


def kernel(x, pe):
    n, two = x.shape
    max_len, d_half = pe.shape

    xt = x.T.reshape(two, n // 128, 128)
    idx3d = _compute_indices(xt, float(max_len - 1), d_half)
    idx_flat = idx3d.reshape(two * n)

    return _make_sc_gather(max_len, d_half, n)(pe.reshape(-1), idx_flat)
